# Initial kernel scaffold; baseline (speedup 1.0000x reference)
import jax
import jax.numpy as jnp
from jax.experimental import pallas as pl


def kernel(src, W_qk, W_v, W_o, b_o, g1, be1, g2, be2, W1, bf1, W2, bf2, rotations):
    raise NotImplementedError("write your pallas kernel here")


# speedup vs baseline: 1.0427x; 1.0427x over previous
import functools

import jax
import jax.numpy as jnp
from jax.experimental import pallas as pl
from jax.experimental.pallas import tpu as pltpu

B, T, D = 1, 4096, 768
H = 12
DH = D // H
N_HASHES = 4
BUCKET_SIZE = 64
N_BUCKETS = T // BUCKET_SIZE
D_FFN = 3072
EPS = 1e-6
SELF_ATTN_VALUE = -5e4
NEG_INF = -1e30

TB = 512
N_TBLK = T // TB
CHUNK = BUCKET_SIZE
N_CHUNKS = N_HASHES * N_BUCKETS
G = 4
QB = G * CHUNK
KB = QB + CHUNK
N_JBLK = N_CHUNKS // G


def _pre_body(x_ref, wqk_ref, wv_ref, g_ref, b_ref, rot_ref,
              qkh_ref, vh_ref, bkt_ref):
    x = x_ref[...]
    m = jnp.mean(x, axis=-1, keepdims=True)
    xc = x - m
    var = jnp.mean(xc * xc, axis=-1, keepdims=True)
    h = xc * jax.lax.rsqrt(var + EPS) * g_ref[0] + b_ref[0]
    qk = jnp.dot(h, wqk_ref[...], preferred_element_type=jnp.float32)
    vv = jnp.dot(h, wv_ref[...], preferred_element_type=jnp.float32)
    rot = rot_ref[...]
    for hd in range(H):
        qh = qk[:, hd * DH:(hd + 1) * DH]
        qkh_ref[hd] = qh
        vh_ref[hd] = vv[:, hd * DH:(hd + 1) * DH]
        r = jnp.dot(qh, rot, preferred_element_type=jnp.float32)
        for i in range(N_HASHES):
            ri = r[:, i * (N_BUCKETS // 2):(i + 1) * (N_BUCKETS // 2)]
            full = jnp.concatenate([ri, -ri], axis=-1)
            bkt_ref[hd * N_HASHES + i] = jnp.argmax(full, axis=-1).astype(jnp.int32)


def _pre_call(x, W_qk, W_v, g1, be1, rot2):
    return pl.pallas_call(
        _pre_body,
        grid=(N_TBLK,),
        in_specs=[
            pl.BlockSpec((TB, D), lambda j: (j, 0)),
            pl.BlockSpec((D, D), lambda j: (0, 0)),
            pl.BlockSpec((D, D), lambda j: (0, 0)),
            pl.BlockSpec((1, D), lambda j: (0, 0)),
            pl.BlockSpec((1, D), lambda j: (0, 0)),
            pl.BlockSpec((DH, N_HASHES * N_BUCKETS // 2), lambda j: (0, 0)),
        ],
        out_specs=[
            pl.BlockSpec((H, TB, DH), lambda j: (0, j, 0)),
            pl.BlockSpec((H, TB, DH), lambda j: (0, j, 0)),
            pl.BlockSpec((H * N_HASHES, TB), lambda j: (0, j)),
        ],
        out_shape=[
            jax.ShapeDtypeStruct((H, T, DH), jnp.float32),
            jax.ShapeDtypeStruct((H, T, DH), jnp.float32),
            jax.ShapeDtypeStruct((H * N_HASHES, T), jnp.int32),
        ],
    )(x, W_qk, W_v, g1, be1, rot2)


def _attn_body(q_ref, pq_ref, v_ref, pv_ref, tq_ref, tp_ref, out_ref):
    q = q_ref[0].reshape(QB, DH)
    pq = pq_ref[0, 0]
    kraw = jnp.concatenate([pq, q], axis=0)
    norm = jnp.sqrt(jnp.sum(kraw * kraw, axis=-1, keepdims=True))
    bk = kraw / jnp.maximum(norm, 1e-12)
    v = v_ref[0].reshape(QB, DH)
    bv = jnp.concatenate([pv_ref[0, 0], v], axis=0)

    dots = jax.lax.dot_general(q, bk, (((1,), (1,)), ((), ())),
                               preferred_element_type=jnp.float32)
    dots = dots * (DH ** -0.5)

    qt = tq_ref[0, 0, 0]
    kt = jnp.concatenate([tp_ref[0, 0, 0], qt], axis=0)
    ri = jax.lax.broadcasted_iota(jnp.int32, (QB, KB), 0)
    ci = jax.lax.broadcasted_iota(jnp.int32, (QB, KB), 1)
    lo = (ri // CHUNK) * CHUNK
    band = (ci >= lo) & (ci < lo + 2 * CHUNK)
    selfm = qt[:, None] == kt[None, :]
    scores = jnp.where(band,
                       jnp.where(selfm, SELF_ATTN_VALUE, dots),
                       NEG_INF)
    m = jnp.max(scores, axis=-1, keepdims=True)
    p = jnp.exp(scores - m)
    s = jnp.sum(p, axis=-1, keepdims=True)
    lse = m + jnp.log(s)
    bo = jnp.dot(p / s, bv, preferred_element_type=jnp.float32)
    out_ref[0, 0] = jnp.concatenate(
        [bo, lse, jnp.zeros((QB, 15), jnp.float32)], axis=-1)


def _attn_call(sqk4, spq4, sv4, spv4, tq4, tp4):
    return pl.pallas_call(
        _attn_body,
        grid=(H, N_JBLK),
        in_specs=[
            pl.BlockSpec((1, G, CHUNK, DH), lambda b, j: (b, j, 0, 0)),
            pl.BlockSpec((1, 1, CHUNK, DH),
                         lambda b, j: (b, (G * j - 1) % N_CHUNKS, 0, 0)),
            pl.BlockSpec((1, G, CHUNK, DH), lambda b, j: (b, j, 0, 0)),
            pl.BlockSpec((1, 1, CHUNK, DH),
                         lambda b, j: (b, (G * j - 1) % N_CHUNKS, 0, 0)),
            pl.BlockSpec((1, 1, 1, QB), lambda b, j: (b, j, 0, 0)),
            pl.BlockSpec((1, 1, 1, CHUNK),
                         lambda b, j: (b, (G * j - 1) % N_CHUNKS, 0, 0)),
        ],
        out_specs=pl.BlockSpec((1, 1, QB, 80), lambda b, j: (b, j, 0, 0)),
        out_shape=jax.ShapeDtypeStruct((H, N_JBLK, QB, 80), jnp.float32),
    )(sqk4, spq4, sv4, spv4, tq4, tp4)


def _post_body(o_ref, x_ref, wo_ref, bo_ref, g_ref, b_ref,
               w1_ref, b1_ref, w2_ref, b2_ref, out_ref):
    parts = []
    for hd in range(H):
        oh = o_ref[hd]
        logits = oh[:, :, 64]
        m = jnp.max(logits, axis=0, keepdims=True)
        p = jnp.exp(logits - m)
        w = p / jnp.sum(p, axis=0, keepdims=True)
        acc = jnp.sum(oh[:, :, :DH] * w[:, :, None], axis=0)
        parts.append(acc)
    attn = jnp.concatenate(parts, axis=-1)
    attn = jnp.dot(attn, wo_ref[...], preferred_element_type=jnp.float32)
    src2 = x_ref[...] + attn + bo_ref[0]
    m2 = jnp.mean(src2, axis=-1, keepdims=True)
    xc = src2 - m2
    var = jnp.mean(xc * xc, axis=-1, keepdims=True)
    src3 = xc * jax.lax.rsqrt(var + EPS) * g_ref[0] + b_ref[0]
    hmid = jnp.maximum(
        jnp.dot(src3, w1_ref[...], preferred_element_type=jnp.float32)
        + b1_ref[0], 0.0)
    ff = jnp.dot(hmid, w2_ref[...], preferred_element_type=jnp.float32) + b2_ref[0]
    out_ref[...] = src2 + ff


def _post_call(o_ext, x, W_o, b_o, g2, be2, W1, bf1, W2, bf2):
    return pl.pallas_call(
        _post_body,
        grid=(N_TBLK,),
        in_specs=[
            pl.BlockSpec((H, N_HASHES, TB, 80), lambda j: (0, 0, j, 0)),
            pl.BlockSpec((TB, D), lambda j: (j, 0)),
            pl.BlockSpec((D, D), lambda j: (0, 0)),
            pl.BlockSpec((1, D), lambda j: (0, 0)),
            pl.BlockSpec((1, D), lambda j: (0, 0)),
            pl.BlockSpec((1, D), lambda j: (0, 0)),
            pl.BlockSpec((D, D_FFN), lambda j: (0, 0)),
            pl.BlockSpec((1, D_FFN), lambda j: (0, 0)),
            pl.BlockSpec((D_FFN, D), lambda j: (0, 0)),
            pl.BlockSpec((1, D), lambda j: (0, 0)),
        ],
        out_specs=pl.BlockSpec((TB, D), lambda j: (j, 0)),
        out_shape=jax.ShapeDtypeStruct((T, D), jnp.float32),
    )(o_ext, x, W_o, b_o, g2, be2, W1, bf1, W2, bf2)


@jax.jit
def _run(src, W_qk, W_v, W_o, b_o, g1, be1, g2, be2, W1, bf1, W2, bf2,
         rotations):
    x = src.reshape(T, D)
    rot2 = rotations.reshape(DH, N_HASHES * (N_BUCKETS // 2))
    qkh, vh, bkt = _pre_call(x, W_qk, W_v, g1.reshape(1, D), be1.reshape(1, D),
                             rot2)

    buckets = bkt.reshape(H, N_HASHES, T)
    keys = buckets * T + jnp.arange(T, dtype=jnp.int32)[None, None, :]
    st = jnp.argsort(keys, axis=-1).astype(jnp.int32)

    sqk = jnp.take_along_axis(qkh[:, None], st[..., None], axis=2)
    sv = jnp.take_along_axis(vh[:, None], st[..., None], axis=2)
    sqk4 = sqk.reshape(H, N_CHUNKS, CHUNK, DH)
    sv4 = sv.reshape(H, N_CHUNKS, CHUNK, DH)
    st_f = st.astype(jnp.float32)
    tq4 = st_f.reshape(H, N_JBLK, 1, QB)
    tp4 = st_f.reshape(H, N_CHUNKS, 1, CHUNK)

    so = _attn_call(sqk4, sqk4, sv4, sv4, tq4, tp4)
    so = so.reshape(H, N_HASHES, T, 80)

    undo = jnp.argsort(st, axis=-1)
    o_ext = jnp.take_along_axis(so, undo[..., None], axis=2)

    out = _post_call(o_ext, x, W_o, b_o.reshape(1, D), g2.reshape(1, D),
                     be2.reshape(1, D), W1, bf1.reshape(1, D_FFN), W2,
                     bf2.reshape(1, D))
    return out.reshape(B, T, D), jnp.zeros((0,), jnp.float32)


def kernel(src, W_qk, W_v, W_o, b_o, g1, be1, g2, be2, W1, bf1, W2, bf2,
           rotations):
    return _run(src, W_qk, W_v, W_o, b_o, g1, be1, g2, be2, W1, bf1, W2, bf2,
                rotations)
